# baseline (speedup 1.0000x reference)
import functools

import jax
import jax.numpy as jnp
from jax import lax
from jax.experimental import pallas as pl
from jax.experimental.pallas import tpu as pltpu
from jax.experimental.pallas import tpu_sc as plsc

NC = 2
NS = 16
NW = NC * NS

VOCAB = 1000000
MAXLEN = 200
EMBED = 64
BATCH = 4096
SEQ = 200

K = 2
CH = K * SEQ
SUB = 80
NSUB = CH // SUB
ROWS_PER_W = BATCH * SEQ // NW
N_CHUNKS = ROWS_PER_W // CH
VREGS = EMBED // 16


def _emb_body(x_hbm, tok_hbm, pos_hbm, out_hbm, pos_v, idx_v, rows_v, sem):
    wid = lax.axis_index("s") * NC + lax.axis_index("c")

    pltpu.sync_copy(pos_hbm, pos_v)

    def chunk(g, carry):
        cid = wid * N_CHUNKS + g
        off = cid * CH

        pltpu.sync_copy(x_hbm.at[cid], idx_v)
        for j in range(NSUB):
            pltpu.async_copy(
                tok_hbm.at[idx_v.at[j]],
                rows_v.at[pl.ds(j * SUB, SUB)],
                sem,
            )
        for j in range(NSUB):
            pltpu.make_async_copy(
                tok_hbm.at[idx_v.at[j]],
                rows_v.at[pl.ds(j * SUB, SUB)],
                sem,
            ).wait()

        def body(s, c):
            for d in range(VREGS):
                pv = pos_v[s, pl.ds(d * 16, 16)]
                for k in range(K):
                    r = k * SEQ + s
                    rows_v[r, pl.ds(d * 16, 16)] = (
                        rows_v[r, pl.ds(d * 16, 16)] + pv
                    )
            return c

        lax.fori_loop(0, SEQ, body, 0)

        pltpu.sync_copy(rows_v, out_hbm.at[pl.ds(off, CH)])
        return carry

    lax.fori_loop(0, N_CHUNKS, chunk, 0)


@functools.partial(jax.jit, static_argnames=())
def _emb(x_flat, token_table, pos_table):
    kern = pl.kernel(
        _emb_body,
        out_type=jax.ShapeDtypeStruct((BATCH * SEQ, EMBED), jnp.float32),
        mesh=plsc.VectorSubcoreMesh(core_axis_name="c", subcore_axis_name="s"),
        scratch_types=[
            pltpu.VMEM((MAXLEN, EMBED), jnp.float32),
            pltpu.VMEM((NSUB, SUB), jnp.int32),
            pltpu.VMEM((CH, EMBED), jnp.float32),
            pltpu.SemaphoreType.DMA,
        ],
        compiler_params=pltpu.CompilerParams(use_tc_tiling_on_sc=False),
    )
    return kern(x_flat, token_table, pos_table)


def kernel(x, token_table, pos_table):
    x_flat = x.astype(jnp.int32).reshape(NW * N_CHUNKS, NSUB, SUB)
    out = _emb(x_flat, token_table, pos_table)
    return out.reshape(BATCH, SEQ, EMBED)


# speedup vs baseline: 1.1166x; 1.1166x over previous
import functools

import jax
import jax.numpy as jnp
from jax import lax
from jax.experimental import pallas as pl
from jax.experimental.pallas import tpu as pltpu
from jax.experimental.pallas import tpu_sc as plsc

NC = 2
NS = 16
NW = NC * NS

VOCAB = 1000000
MAXLEN = 200
EMBED = 64
BATCH = 4096
SEQ = 200

K = 2
CH = K * SEQ
SUB = 80
NSUB = CH // SUB
ROWS_PER_W = BATCH * SEQ // NW
N_CHUNKS = ROWS_PER_W // CH
VREGS = EMBED // 16


def _emb_body(x_hbm, tok_hbm, pos_hbm, out_hbm,
              pos_v, idx_all, rows_a, rows_b, gsem_a, gsem_b, outsem):
    wid = lax.axis_index("s") * NC + lax.axis_index("c")
    base = wid * N_CHUNKS

    pltpu.sync_copy(pos_hbm, pos_v)
    pltpu.sync_copy(x_hbm.at[wid], idx_all)

    def fire(g, rows, sem):
        for j in range(NSUB):
            pltpu.async_copy(
                tok_hbm.at[idx_all.at[g, j]],
                rows.at[pl.ds(j * SUB, SUB)],
                sem,
            )

    def drain(rows, sem):
        pltpu.make_async_copy(tok_hbm.at[pl.ds(0, CH)], rows, sem).wait()

    def add(rows):
        def body(s):
            for d in range(VREGS):
                pv = pos_v[s, pl.ds(d * 16, 16)]
                for k in range(K):
                    r = k * SEQ + s
                    rows[r, pl.ds(d * 16, 16)] = (
                        rows[r, pl.ds(d * 16, 16)] + pv
                    )
        plsc.parallel_loop(0, SEQ, unroll=2)(body)

    def put(g, rows):
        pltpu.async_copy(rows, out_hbm.at[pl.ds((base + g) * CH, CH)], outsem)

    def wait_out():
        pltpu.make_async_copy(rows_a, out_hbm.at[pl.ds(0, CH)], outsem).wait()

    fire(0, rows_a, gsem_a)
    fire(1, rows_b, gsem_b)
    drain(rows_a, gsem_a)
    add(rows_a)
    put(0, rows_a)

    def body(g2, c):
        g = 1 + 2 * g2
        wait_out()
        fire(g + 1, rows_a, gsem_a)
        drain(rows_b, gsem_b)
        add(rows_b)
        put(g, rows_b)
        wait_out()
        fire(g + 2, rows_b, gsem_b)
        drain(rows_a, gsem_a)
        add(rows_a)
        put(g + 1, rows_a)
        return c

    lax.fori_loop(0, (N_CHUNKS - 2) // 2, body, 0)

    drain(rows_b, gsem_b)
    add(rows_b)
    put(N_CHUNKS - 1, rows_b)
    wait_out()
    wait_out()


@jax.jit
def _emb(x_flat, token_table, pos_table):
    kern = pl.kernel(
        _emb_body,
        out_type=jax.ShapeDtypeStruct((BATCH * SEQ, EMBED), jnp.float32),
        mesh=plsc.VectorSubcoreMesh(core_axis_name="c", subcore_axis_name="s"),
        scratch_types=[
            pltpu.VMEM((MAXLEN, EMBED), jnp.float32),
            pltpu.VMEM((N_CHUNKS, NSUB, SUB), jnp.int32),
            pltpu.VMEM((CH, EMBED), jnp.float32),
            pltpu.VMEM((CH, EMBED), jnp.float32),
            pltpu.SemaphoreType.DMA,
            pltpu.SemaphoreType.DMA,
            pltpu.SemaphoreType.DMA,
        ],
        compiler_params=pltpu.CompilerParams(use_tc_tiling_on_sc=False),
    )
    return kern(x_flat, token_table, pos_table)


def kernel(x, token_table, pos_table):
    x_flat = x.astype(jnp.int32).reshape(NW, N_CHUNKS, NSUB, SUB)
    out = _emb(x_flat, token_table, pos_table)
    return out.reshape(BATCH, SEQ, EMBED)


# speedup vs baseline: 1.1180x; 1.0013x over previous
import jax
import jax.numpy as jnp
from jax import lax
from jax.experimental import pallas as pl
from jax.experimental.pallas import tpu as pltpu
from jax.experimental.pallas import tpu_sc as plsc

NC = 2
NS = 16
NW = NC * NS

VOCAB = 1000000
MAXLEN = 200
EMBED = 64
BATCH = 4096
SEQ = 200

K = 2
B_PER_W = BATCH // NW
N_CHUNKS = B_PER_W // K
RUNS = ((0, 128), (128, 72))
VREGS = EMBED // 16


def _emb_body(x_hbm, tok_hbm, pos_hbm, out_hbm,
              pos_v, idx_a, idx_b, rows_a, rows_b,
              isem_a, isem_b, gsem_a, gsem_b, outsem):
    wid = lax.axis_index("s") * NC + lax.axis_index("c")
    base_b = wid * B_PER_W

    pltpu.sync_copy(pos_hbm, pos_v)

    def prefetch(g, idx, isem):
        b0 = base_b + g * K
        pltpu.async_copy(x_hbm.at[pl.ds(b0, K)], idx, isem)

    def launch(g, idx, rows, isem, gsem):
        pltpu.make_async_copy(x_hbm.at[pl.ds(0, K)], idx, isem).wait()
        for k in range(K):
            for off, n in RUNS:
                pltpu.async_copy(
                    tok_hbm.at[idx.at[k, pl.ds(off, n)]],
                    rows.at[k, pl.ds(off, n)],
                    gsem,
                )

    def drain(rows, gsem):
        pltpu.make_async_copy(out_hbm.at[pl.ds(0, K)], rows, gsem).wait()

    def add(rows):
        def body(s):
            for d in range(VREGS):
                pv = pos_v[s, pl.ds(d * 16, 16)]
                for k in range(K):
                    rows[k, s, pl.ds(d * 16, 16)] = (
                        rows[k, s, pl.ds(d * 16, 16)] + pv
                    )
        plsc.parallel_loop(0, SEQ, unroll=2)(body)

    def put(g, rows):
        b0 = base_b + g * K
        pltpu.async_copy(rows, out_hbm.at[pl.ds(b0, K)], outsem)

    def wait_out():
        pltpu.make_async_copy(rows_a, out_hbm.at[pl.ds(0, K)], outsem).wait()

    prefetch(0, idx_a, isem_a)
    prefetch(1, idx_b, isem_b)
    launch(0, idx_a, rows_a, isem_a, gsem_a)
    launch(1, idx_b, rows_b, isem_b, gsem_b)
    drain(rows_a, gsem_a)
    prefetch(2, idx_a, isem_a)
    add(rows_a)
    put(0, rows_a)

    def body(g2, c):
        g = 1 + 2 * g2
        wait_out()
        launch(g + 1, idx_a, rows_a, isem_a, gsem_a)
        drain(rows_b, gsem_b)
        prefetch(g + 2, idx_b, isem_b)
        add(rows_b)
        put(g, rows_b)
        wait_out()
        launch(g + 2, idx_b, rows_b, isem_b, gsem_b)
        drain(rows_a, gsem_a)
        prefetch(jnp.minimum(g + 3, N_CHUNKS - 1), idx_a, isem_a)
        add(rows_a)
        put(g + 1, rows_a)
        return c

    lax.fori_loop(0, (N_CHUNKS - 2) // 2, body, 0)

    drain(rows_b, gsem_b)
    add(rows_b)
    put(N_CHUNKS - 1, rows_b)
    pltpu.make_async_copy(x_hbm.at[pl.ds(0, K)], idx_a, isem_a).wait()
    wait_out()
    wait_out()


@jax.jit
def _emb(x, token_table, pos_table):
    kern = pl.kernel(
        _emb_body,
        out_type=jax.ShapeDtypeStruct((BATCH, SEQ, EMBED), jnp.float32),
        mesh=plsc.VectorSubcoreMesh(core_axis_name="c", subcore_axis_name="s"),
        scratch_types=[
            pltpu.VMEM((MAXLEN, EMBED), jnp.float32),
            pltpu.VMEM((K, SEQ), jnp.int32),
            pltpu.VMEM((K, SEQ), jnp.int32),
            pltpu.VMEM((K, SEQ, EMBED), jnp.float32),
            pltpu.VMEM((K, SEQ, EMBED), jnp.float32),
            pltpu.SemaphoreType.DMA,
            pltpu.SemaphoreType.DMA,
            pltpu.SemaphoreType.DMA,
            pltpu.SemaphoreType.DMA,
            pltpu.SemaphoreType.DMA,
        ],
        compiler_params=pltpu.CompilerParams(use_tc_tiling_on_sc=False),
    )
    return kern(x, token_table, pos_table)


def kernel(x, token_table, pos_table):
    return _emb(x.astype(jnp.int32), token_table, pos_table)
